# baseline (device time: 28532 ns/iter reference)
import jax
import jax.numpy as jnp
from jax import lax
from jax.experimental import pallas as pl
from jax.experimental.pallas import tpu as pltpu

N_DEV = 32
N_ROUNDS = 5


def kernel(x, router_W, route_idx, expert_W, shared_W):
    n, d = x.shape
    h = expert_W.shape[-1]

    def body(x_ref, rw_ref, idx_ref, ew_ref, sw_ref, out_ref,
             acc_ref, recv_buf, send_sems, recv_sems):
        my_i = lax.axis_index("i")

        def partner_for_round(r, p):
            if r == 0:
                return p ^ 1
            if r == 2:
                return p ^ 4
            if r == 3:
                return p ^ 8
            if r == 4:
                return p ^ 16
            z = p >> 3
            s = p & 7
            y = s >> 1
            px = (s & 1) ^ (y & 1)
            ny = y ^ 1
            ns = (ny << 1) | (px ^ (ny & 1))
            return (z << 3) | ns

        barrier_sem = pltpu.get_barrier_semaphore()
        for r in range(N_ROUNDS):
            pl.semaphore_signal(
                barrier_sem, inc=1,
                device_id=(partner_for_round(r, my_i),),
                device_id_type=pl.DeviceIdType.MESH,
            )

        xv = x_ref[:, :]
        scores = jnp.dot(xv, rw_ref[:, :], preferred_element_type=jnp.float32)
        s_max = jnp.max(scores, axis=-1, keepdims=True)
        e = jnp.exp(scores - s_max)
        probs = e / jnp.sum(e, axis=-1, keepdims=True)
        idx = idx_ref[:, :]
        eids = lax.broadcasted_iota(jnp.int32, scores.shape, 1)
        p_sel = jnp.sum(jnp.where(eids == idx, probs, 0.0),
                        axis=-1, keepdims=True)
        w0 = jnp.where(idx == 2 * my_i, p_sel, 0.0)
        w1 = jnp.where(idx == 2 * my_i + 1, p_sel, 0.0)
        y0 = jnp.dot(xv, ew_ref[0], preferred_element_type=jnp.float32)
        y1 = jnp.dot(xv, ew_ref[1], preferred_element_type=jnp.float32)
        acc_ref[:, :] = (w0 * y0 + w1 * y1).astype(jnp.bfloat16)
        pl.semaphore_wait(barrier_sem, N_ROUNDS)

        for t in range(N_ROUNDS):
            rdma = pltpu.make_async_remote_copy(
                src_ref=acc_ref,
                dst_ref=recv_buf.at[t],
                send_sem=send_sems.at[t],
                recv_sem=recv_sems.at[t],
                device_id=(partner_for_round(t, my_i),),
                device_id_type=pl.DeviceIdType.MESH,
            )
            rdma.start()
            if t == 0:
                out_ref[:, :] = jnp.dot(
                    xv, sw_ref[:, :], preferred_element_type=jnp.float32)
            rdma.wait()
            if t < N_ROUNDS - 1:
                acc_ref[:, :] = acc_ref[:, :] + recv_buf[t]
            else:
                out_ref[:, :] = out_ref[:, :] + (
                    acc_ref[:, :] + recv_buf[t]).astype(jnp.float32)

    return pl.pallas_call(
        body,
        out_shape=jax.ShapeDtypeStruct((n, h), jnp.float32),
        in_specs=[pl.BlockSpec(memory_space=pltpu.VMEM)] * 5,
        out_specs=pl.BlockSpec(memory_space=pltpu.VMEM),
        scratch_shapes=[
            pltpu.VMEM((n, h), jnp.bfloat16),
            pltpu.VMEM((N_ROUNDS, n, h), jnp.bfloat16),
            pltpu.SemaphoreType.DMA((N_ROUNDS,)),
            pltpu.SemaphoreType.DMA((N_ROUNDS,)),
        ],
        compiler_params=pltpu.CompilerParams(collective_id=0),
    )(x, router_W, route_idx, expert_W, shared_W)


# device time: 23109 ns/iter; 1.2347x vs baseline; 1.2347x over previous
import jax
import jax.numpy as jnp
from jax import lax
from jax.experimental import pallas as pl
from jax.experimental.pallas import tpu as pltpu

N_DEV = 32
N_ROUNDS = 5
N_CHAINS = 4


def kernel(x, router_W, route_idx, expert_W, shared_W):
    n, d = x.shape
    h = expert_W.shape[-1]
    q = n // N_CHAINS

    def body(x_ref, rw_ref, idx_ref, ew_ref, sw_ref, out_ref,
             acc_ref, recv_buf, send_sems, recv_sems):
        my_i = lax.axis_index("i")

        def partner_for_round(r, p):
            if r == 0:
                return p ^ 1
            if r == 2:
                return p ^ 4
            if r == 3:
                return p ^ 8
            if r == 4:
                return p ^ 16
            z = p >> 3
            s = p & 7
            y = s >> 1
            px = (s & 1) ^ (y & 1)
            ny = y ^ 1
            ns = (ny << 1) | (px ^ (ny & 1))
            return (z << 3) | ns

        barrier_sem = pltpu.get_barrier_semaphore()
        for r in range(N_ROUNDS):
            pl.semaphore_signal(
                barrier_sem, inc=1,
                device_id=(partner_for_round(r, my_i),),
                device_id_type=pl.DeviceIdType.MESH,
            )

        xv = x_ref[:, :]
        scores = jnp.dot(xv, rw_ref[:, :], preferred_element_type=jnp.float32)
        s_max = jnp.max(scores, axis=-1, keepdims=True)
        e = jnp.exp(scores - s_max)
        probs = e / jnp.sum(e, axis=-1, keepdims=True)
        idx = idx_ref[:, :]
        eids = lax.broadcasted_iota(jnp.int32, scores.shape, 1)
        p_sel = jnp.sum(jnp.where(eids == idx, probs, 0.0),
                        axis=-1, keepdims=True)
        w0 = jnp.where(idx == 2 * my_i, p_sel, 0.0)
        w1 = jnp.where(idx == 2 * my_i + 1, p_sel, 0.0)

        def partial_q(c):
            lo = c * q
            xh = xv[lo:lo + q, :]
            y0 = jnp.dot(xh, ew_ref[0], preferred_element_type=jnp.float32)
            y1 = jnp.dot(xh, ew_ref[1], preferred_element_type=jnp.float32)
            return (w0[lo:lo + q] * y0 + w1[lo:lo + q] * y1).astype(
                jnp.bfloat16)

        for t in range(N_ROUNDS):
            rds = [
                pltpu.make_async_remote_copy(
                    src_ref=acc_ref.at[pl.ds(c * q, q)],
                    dst_ref=recv_buf.at[c, t],
                    send_sem=send_sems.at[c, t],
                    recv_sem=recv_sems.at[c, t],
                    device_id=(partner_for_round((t + c) % N_ROUNDS, my_i),),
                    device_id_type=pl.DeviceIdType.MESH,
                )
                for c in range(N_CHAINS)
            ]
            if t == 0:
                acc_ref[pl.ds(0, q), :] = partial_q(0)
                pl.semaphore_wait(barrier_sem, N_ROUNDS)
                rds[0].start()
                for c in range(1, N_CHAINS):
                    acc_ref[pl.ds(c * q, q), :] = partial_q(c)
                    rds[c].start()
                out_ref[:, :] = jnp.dot(
                    xv, sw_ref[:, :], preferred_element_type=jnp.float32)
            else:
                for rd in rds:
                    rd.start()
            for c, rd in enumerate(rds):
                rd.wait()
                if t < N_ROUNDS - 1:
                    acc_ref[pl.ds(c * q, q), :] = (
                        acc_ref[pl.ds(c * q, q), :] + recv_buf[c, t])
                else:
                    out_ref[pl.ds(c * q, q), :] = (
                        out_ref[pl.ds(c * q, q), :]
                        + (acc_ref[pl.ds(c * q, q), :]
                           + recv_buf[c, t]).astype(jnp.float32))

    return pl.pallas_call(
        body,
        out_shape=jax.ShapeDtypeStruct((n, h), jnp.float32),
        in_specs=[pl.BlockSpec(memory_space=pltpu.VMEM)] * 5,
        out_specs=pl.BlockSpec(memory_space=pltpu.VMEM),
        scratch_shapes=[
            pltpu.VMEM((n, h), jnp.bfloat16),
            pltpu.VMEM((N_CHAINS, N_ROUNDS, q, h), jnp.bfloat16),
            pltpu.SemaphoreType.DMA((N_CHAINS, N_ROUNDS)),
            pltpu.SemaphoreType.DMA((N_CHAINS, N_ROUNDS)),
        ],
        compiler_params=pltpu.CompilerParams(collective_id=0),
    )(x, router_W, route_idx, expert_W, shared_W)


# device time: 22712 ns/iter; 1.2563x vs baseline; 1.0175x over previous
import jax
import jax.numpy as jnp
from jax import lax
from jax.experimental import pallas as pl
from jax.experimental.pallas import tpu as pltpu

N_DEV = 32
N_ROUNDS = 5
E_PER_DEV = 2


def kernel(x, router_W, route_idx, expert_W, shared_W):
    n, d = x.shape
    h = expert_W.shape[-1]

    half = n // 2

    def body(x_ref, rw_ref, idx_ref, ew_ref, sw_ref, out_ref,
             acc_ref, recv_a, recv_b,
             send_sems_a, recv_sems_a, send_sems_b, recv_sems_b):
        my_i = lax.axis_index("i")

        def partner_for_round(r, p):
            if r == 0:
                return p ^ 1
            if r == 2:
                return p ^ 4
            if r == 3:
                return p ^ 8
            if r == 4:
                return p ^ 16
            z = p >> 3
            s = p & 7
            y = s >> 1
            x = (s & 1) ^ (y & 1)
            ny = y ^ 1
            ns = (ny << 1) | (x ^ (ny & 1))
            return (z << 3) | ns

        barrier_sem = pltpu.get_barrier_semaphore()
        for r in range(N_ROUNDS):
            pl.semaphore_signal(
                barrier_sem, inc=1,
                device_id=(partner_for_round(r, my_i),),
                device_id_type=pl.DeviceIdType.MESH,
            )

        xv = x_ref[:, :]

        scores = jnp.dot(xv, rw_ref[:, :], preferred_element_type=jnp.float32)
        s_max = jnp.max(scores, axis=-1, keepdims=True)
        e = jnp.exp(scores - s_max)
        probs = e / jnp.sum(e, axis=-1, keepdims=True)
        idx = idx_ref[:, :]
        eids = lax.broadcasted_iota(jnp.int32, scores.shape, 1)
        p_sel = jnp.sum(jnp.where(eids == idx, probs, 0.0),
                        axis=-1, keepdims=True)

        w0 = jnp.where(idx == 2 * my_i, p_sel, 0.0)
        w1 = jnp.where(idx == 2 * my_i + 1, p_sel, 0.0)

        def partial_half(lo):
            xh = xv[lo:lo + half, :]
            y0 = jnp.dot(xh, ew_ref[0], preferred_element_type=jnp.float32)
            y1 = jnp.dot(xh, ew_ref[1], preferred_element_type=jnp.float32)
            return (w0[lo:lo + half] * y0
                    + w1[lo:lo + half] * y1).astype(jnp.bfloat16)

        A_ORDER = [2, 1, 0, 3, 4]
        B_ORDER = [4, 3, 1, 0, 2]
        for t in range(N_ROUNDS):
            pa = partner_for_round(A_ORDER[t], my_i)
            pb = partner_for_round(B_ORDER[t], my_i)
            rdma_a = pltpu.make_async_remote_copy(
                src_ref=acc_ref.at[pl.ds(0, half)],
                dst_ref=recv_a.at[t],
                send_sem=send_sems_a.at[t],
                recv_sem=recv_sems_a.at[t],
                device_id=(pa,),
                device_id_type=pl.DeviceIdType.MESH,
            )
            rdma_b = pltpu.make_async_remote_copy(
                src_ref=acc_ref.at[pl.ds(half, half)],
                dst_ref=recv_b.at[t],
                send_sem=send_sems_b.at[t],
                recv_sem=recv_sems_b.at[t],
                device_id=(pb,),
                device_id_type=pl.DeviceIdType.MESH,
            )
            if t == 0:
                acc_ref[pl.ds(0, half), :] = partial_half(0)
                pl.semaphore_wait(barrier_sem, N_ROUNDS)
                rdma_a.start()
                acc_ref[pl.ds(half, half), :] = partial_half(half)
                rdma_b.start()
                out_ref[:, :] = jnp.dot(
                    xv, sw_ref[:, :], preferred_element_type=jnp.float32)
            else:
                rdma_a.start()
                rdma_b.start()
            rdma_a.wait()
            if t < N_ROUNDS - 1:
                acc_ref[pl.ds(0, half), :] = (
                    acc_ref[pl.ds(0, half), :] + recv_a[t])
            else:
                out_ref[pl.ds(0, half), :] = (
                    out_ref[pl.ds(0, half), :]
                    + (acc_ref[pl.ds(0, half), :]
                       + recv_a[t]).astype(jnp.float32))
            rdma_b.wait()
            if t < N_ROUNDS - 1:
                acc_ref[pl.ds(half, half), :] = (
                    acc_ref[pl.ds(half, half), :] + recv_b[t])
            else:
                out_ref[pl.ds(half, half), :] = (
                    out_ref[pl.ds(half, half), :]
                    + (acc_ref[pl.ds(half, half), :]
                       + recv_b[t]).astype(jnp.float32))

    return pl.pallas_call(
        body,
        out_shape=jax.ShapeDtypeStruct((n, h), jnp.float32),
        in_specs=[pl.BlockSpec(memory_space=pltpu.VMEM)] * 5,
        out_specs=pl.BlockSpec(memory_space=pltpu.VMEM),
        scratch_shapes=[
            pltpu.VMEM((n, h), jnp.bfloat16),
            pltpu.VMEM((N_ROUNDS, n // 2, h), jnp.bfloat16),
            pltpu.VMEM((N_ROUNDS, n // 2, h), jnp.bfloat16),
            pltpu.SemaphoreType.DMA((N_ROUNDS,)),
            pltpu.SemaphoreType.DMA((N_ROUNDS,)),
            pltpu.SemaphoreType.DMA((N_ROUNDS,)),
            pltpu.SemaphoreType.DMA((N_ROUNDS,)),
        ],
        compiler_params=pltpu.CompilerParams(collective_id=0),
    )(x, router_W, route_idx, expert_W, shared_W)
